# baseline (speedup 1.0000x reference)
import functools
import math

import jax
import jax.numpy as jnp
from jax.experimental import pallas as pl
from jax.experimental.pallas import tpu as pltpu

_V = 32000
_PAD = 0
_SMOOTHING = 0.1
_CONF = 1.0 - _SMOOTHING
_EPS = _SMOOTHING / (_V - 1)
_CS = _CONF - _EPS
_C0 = (_V - 1) * _EPS * math.log(_EPS) + _CONF * math.log(_CONF)

_R = 256
_C = 16000


def _body(nc, inv_denom, x_ref, t_ref, out_ref, m_ref, s_ref, sx_ref, xt_ref):
    j = pl.program_id(1)
    xb = x_ref[...]
    t = t_ref[...]

    bmax = jnp.max(xb, axis=1, keepdims=True)
    ones = jnp.ones((_C, 1), dtype=jnp.float32)
    bsum = jax.lax.dot_general(xb, ones, (((1,), (0,)), ((), ())),
                               preferred_element_type=jnp.float32)

    ids = j * _C + jax.lax.broadcasted_iota(jnp.int32, (_R, _C), 1)
    hit = ids == t
    xt_part = jnp.sum(jnp.where(hit, xb, 0.0), axis=1, keepdims=True)

    first = j == 0
    neg_inf = jnp.full((_R, 1), -jnp.inf, dtype=jnp.float32)
    zeros = jnp.zeros((_R, 1), dtype=jnp.float32)
    m_old = jnp.where(first, neg_inf, m_ref[...])
    s_old = jnp.where(first, zeros, s_ref[...])
    sx_old = jnp.where(first, zeros, sx_ref[...])
    xt_old = jnp.where(first, zeros, xt_ref[...])

    m_new = jnp.maximum(m_old, bmax)
    s_new = s_old * jnp.exp(m_old - m_new) + jnp.sum(
        jnp.exp(xb - m_new), axis=1, keepdims=True)
    m_ref[...] = m_new
    s_ref[...] = s_new
    sx_ref[...] = sx_old + bsum
    xt_ref[...] = xt_old + xt_part

    @pl.when(j == nc - 1)
    def _():
        lse = m_new + jnp.log(s_new)
        row_loss = _C0 + lse - _EPS * sx_ref[...] - _CS * xt_ref[...]
        valid = t != _PAD
        contrib = jnp.sum(jnp.where(valid, row_loss, 0.0)) * inv_denom
        out_ref[...] = jnp.zeros((1, 1, 1), jnp.float32) + contrib


def kernel(x, target):
    batch = x.shape[0]
    n = x.shape[0] * x.shape[1]
    xf = x.reshape(n, _V)
    t = target.reshape(n, 1).astype(jnp.int32)
    nr = n // _R
    nc = _V // _C
    out = pl.pallas_call(
        functools.partial(_body, nc, 1.0 / batch),
        grid=(nr, nc),
        in_specs=[
            pl.BlockSpec((_R, _C), lambda i, j: (i, j)),
            pl.BlockSpec((_R, 1), lambda i, j: (i, 0)),
        ],
        out_specs=pl.BlockSpec((1, 1, 1), lambda i, j: (i, 0, 0)),
        out_shape=jax.ShapeDtypeStruct((nr, 1, 1), jnp.float32),
        scratch_shapes=[
            pltpu.VMEM((_R, 1), jnp.float32),
            pltpu.VMEM((_R, 1), jnp.float32),
            pltpu.VMEM((_R, 1), jnp.float32),
            pltpu.VMEM((_R, 1), jnp.float32),
        ],
        compiler_params=pltpu.CompilerParams(
            dimension_semantics=("parallel", "arbitrary"),
        ),
    )(xf, t)
    return jnp.sum(out)


# speedup vs baseline: 1.0427x; 1.0427x over previous
import functools
import math

import jax
import jax.numpy as jnp
from jax.experimental import pallas as pl
from jax.experimental.pallas import tpu as pltpu

_V = 32000
_PAD = 0
_SMOOTHING = 0.1
_CONF = 1.0 - _SMOOTHING
_EPS = _SMOOTHING / (_V - 1)
_CS = _CONF - _EPS
_C0 = (_V - 1) * _EPS * math.log(_EPS) + _CONF * math.log(_CONF)

_R = 256
_C = 16000


def _gather_dma(xany_ref, gbuf, sem, t_sref, i, r):
    row = i * _R + r
    tv = t_sref[row]
    col = (tv >> 7) * 128
    return pltpu.make_async_copy(
        xany_ref.at[pl.ds(row, 1), pl.ds(col, 128)],
        gbuf.at[pl.ds(r, 1)],
        sem,
    )


def _body(nc, inv_denom, t_sref, x_ref, t_ref, xany_ref, out_ref,
          m_ref, s_ref, sx_ref, gbuf, sem):
    i = pl.program_id(0)
    j = pl.program_id(1)
    xb = x_ref[...]
    t = t_ref[...]

    @pl.when(j == 0)
    def _():
        for r in range(_R):
            _gather_dma(xany_ref, gbuf, sem, t_sref, i, r).start()

    bmax = jnp.max(xb, axis=1, keepdims=True)
    bsum = jnp.sum(xb, axis=1, keepdims=True)

    first = j == 0
    neg_inf = jnp.full((_R, 1), -jnp.inf, dtype=jnp.float32)
    zeros = jnp.zeros((_R, 1), dtype=jnp.float32)
    m_old = jnp.where(first, neg_inf, m_ref[...])
    s_old = jnp.where(first, zeros, s_ref[...])
    sx_old = jnp.where(first, zeros, sx_ref[...])

    m_new = jnp.maximum(m_old, bmax)
    s_new = s_old * jnp.exp(m_old - m_new) + jnp.sum(
        jnp.exp(xb - m_new), axis=1, keepdims=True)
    m_ref[...] = m_new
    s_ref[...] = s_new
    sx_ref[...] = sx_old + bsum

    @pl.when(j == nc - 1)
    def _():
        for r in range(_R):
            _gather_dma(xany_ref, gbuf, sem, t_sref, i, r).wait()
        g = gbuf[...]
        li = jax.lax.broadcasted_iota(jnp.int32, (_R, 128), 1)
        lane_t = jnp.bitwise_and(t, 127)
        xt = jnp.sum(jnp.where(li == lane_t, g, 0.0), axis=1, keepdims=True)
        lse = m_new + jnp.log(s_new)
        row_loss = _C0 + lse - _EPS * sx_ref[...] - _CS * xt
        valid = t != _PAD
        contrib = jnp.sum(jnp.where(valid, row_loss, 0.0)) * inv_denom
        out_ref[...] = jnp.zeros((1, 1, 1), jnp.float32) + contrib


def kernel(x, target):
    batch = x.shape[0]
    n = x.shape[0] * x.shape[1]
    xf = x.reshape(n, _V)
    t = target.reshape(n).astype(jnp.int32)
    nr = n // _R
    nc = _V // _C
    out = pl.pallas_call(
        functools.partial(_body, nc, 1.0 / batch),
        grid_spec=pltpu.PrefetchScalarGridSpec(
            num_scalar_prefetch=1,
            grid=(nr, nc),
            in_specs=[
                pl.BlockSpec((_R, _C), lambda i, j, ts: (i, j)),
                pl.BlockSpec((_R, 1), lambda i, j, ts: (i, 0)),
                pl.BlockSpec(memory_space=pl.ANY),
            ],
            out_specs=pl.BlockSpec((1, 1, 1), lambda i, j, ts: (i, 0, 0)),
            scratch_shapes=[
                pltpu.VMEM((_R, 1), jnp.float32),
                pltpu.VMEM((_R, 1), jnp.float32),
                pltpu.VMEM((_R, 1), jnp.float32),
                pltpu.VMEM((_R, 128), jnp.float32),
                pltpu.SemaphoreType.DMA,
            ],
        ),
        out_shape=jax.ShapeDtypeStruct((nr, 1, 1), jnp.float32),
        compiler_params=pltpu.CompilerParams(
            dimension_semantics=("arbitrary", "arbitrary"),
        ),
    )(t, xf, t.reshape(n, 1), xf)
    return jnp.sum(out)


# speedup vs baseline: 1.0437x; 1.0009x over previous
import functools
import math

import jax
import jax.numpy as jnp
from jax.experimental import pallas as pl
from jax.experimental.pallas import tpu as pltpu

_V = 32000
_PAD = 0
_SMOOTHING = 0.1
_CONF = 1.0 - _SMOOTHING
_EPS = _SMOOTHING / (_V - 1)
_CS = _CONF - _EPS
_C0 = (_V - 1) * _EPS * math.log(_EPS) + _CONF * math.log(_CONF)

_R = 512
_C = 6400


def _body(nc, inv_denom, x_ref, t_ref, out_ref, m_ref, s_ref, sx_ref, xt_ref):
    j = pl.program_id(1)
    xb = x_ref[...]
    t = t_ref[...]

    bmax = jnp.max(xb, axis=1, keepdims=True)
    bsum = jnp.sum(xb, axis=1, keepdims=True)

    ids = j * _C + jax.lax.broadcasted_iota(jnp.int32, (_R, _C), 1)
    hit = ids == t
    xt_part = jnp.sum(jnp.where(hit, xb, 0.0), axis=1, keepdims=True)

    first = j == 0
    neg_inf = jnp.full((_R, 1), -jnp.inf, dtype=jnp.float32)
    zeros = jnp.zeros((_R, 1), dtype=jnp.float32)
    m_old = jnp.where(first, neg_inf, m_ref[...])
    s_old = jnp.where(first, zeros, s_ref[...])
    sx_old = jnp.where(first, zeros, sx_ref[...])
    xt_old = jnp.where(first, zeros, xt_ref[...])

    m_new = jnp.maximum(m_old, bmax)
    s_new = s_old * jnp.exp(m_old - m_new) + jnp.sum(
        jnp.exp(xb - m_new), axis=1, keepdims=True)
    m_ref[...] = m_new
    s_ref[...] = s_new
    sx_ref[...] = sx_old + bsum
    xt_ref[...] = xt_old + xt_part

    @pl.when(j == nc - 1)
    def _():
        lse = m_new + jnp.log(s_new)
        row_loss = _C0 + lse - _EPS * sx_ref[...] - _CS * xt_ref[...]
        valid = t != _PAD
        contrib = jnp.sum(jnp.where(valid, row_loss, 0.0)) * inv_denom
        out_ref[...] = jnp.zeros((1, 1, 1), jnp.float32) + contrib


def kernel(x, target):
    batch = x.shape[0]
    n = x.shape[0] * x.shape[1]
    xf = x.reshape(n, _V)
    t = target.reshape(n, 1).astype(jnp.int32)
    nr = n // _R
    nc = _V // _C
    out = pl.pallas_call(
        functools.partial(_body, nc, 1.0 / batch),
        grid=(nr, nc),
        in_specs=[
            pl.BlockSpec((_R, _C), lambda i, j: (i, j)),
            pl.BlockSpec((_R, 1), lambda i, j: (i, 0)),
        ],
        out_specs=pl.BlockSpec((1, 1, 1), lambda i, j: (i, 0, 0)),
        out_shape=jax.ShapeDtypeStruct((nr, 1, 1), jnp.float32),
        scratch_shapes=[
            pltpu.VMEM((_R, 1), jnp.float32),
            pltpu.VMEM((_R, 1), jnp.float32),
            pltpu.VMEM((_R, 1), jnp.float32),
            pltpu.VMEM((_R, 1), jnp.float32),
        ],
        compiler_params=pltpu.CompilerParams(
            dimension_semantics=("parallel", "arbitrary"),
        ),
    )(xf, t)
    return jnp.sum(out)


# speedup vs baseline: 1.1173x; 1.0705x over previous
import functools
import math

import jax
import jax.numpy as jnp
from jax.experimental import pallas as pl
from jax.experimental.pallas import tpu as pltpu

_V = 32000
_PAD = 0
_SMOOTHING = 0.1
_CONF = 1.0 - _SMOOTHING
_EPS = _SMOOTHING / (_V - 1)
_CS = _CONF - _EPS
_C0 = (_V - 1) * _EPS * math.log(_EPS) + _CONF * math.log(_CONF)

_R = 256
_C = 16000


def _body(nc, inv_denom, x_ref, t_ref, out_ref, m_ref, s_ref, sx_ref):
    j = pl.program_id(1)
    xb = x_ref[...]
    t = t_ref[...]

    bmax = jnp.max(xb, axis=1, keepdims=True)

    ids = j * _C + jax.lax.broadcasted_iota(jnp.int32, (_R, _C), 1)
    hit = ids == t
    z_part = jnp.sum(xb * jnp.where(hit, _CONF, _EPS), axis=1, keepdims=True)

    first = j == 0
    neg_inf = jnp.full((_R, 1), -jnp.inf, dtype=jnp.float32)
    zeros = jnp.zeros((_R, 1), dtype=jnp.float32)
    m_old = jnp.where(first, neg_inf, m_ref[...])
    s_old = jnp.where(first, zeros, s_ref[...])
    z_old = jnp.where(first, zeros, sx_ref[...])

    m_new = jnp.maximum(m_old, bmax)
    s_new = s_old * jnp.exp(m_old - m_new) + jnp.sum(
        jnp.exp(xb - m_new), axis=1, keepdims=True)
    m_ref[...] = m_new
    s_ref[...] = s_new
    sx_ref[...] = z_old + z_part

    @pl.when(j == nc - 1)
    def _():
        lse = m_new + jnp.log(s_new)
        row_loss = _C0 + lse - sx_ref[...]
        valid = t != _PAD
        contrib = jnp.sum(jnp.where(valid, row_loss, 0.0)) * inv_denom
        out_ref[...] = jnp.zeros((1, 1, 1), jnp.float32) + contrib


def kernel(x, target):
    batch = x.shape[0]
    n = x.shape[0] * x.shape[1]
    xf = x.reshape(n, _V)
    t = target.reshape(n, 1).astype(jnp.int32)
    nr = n // _R
    nc = _V // _C
    out = pl.pallas_call(
        functools.partial(_body, nc, 1.0 / batch),
        grid=(nr, nc),
        in_specs=[
            pl.BlockSpec((_R, _C), lambda i, j: (i, j)),
            pl.BlockSpec((_R, 1), lambda i, j: (i, 0)),
        ],
        out_specs=pl.BlockSpec((1, 1, 1), lambda i, j: (i, 0, 0)),
        out_shape=jax.ShapeDtypeStruct((nr, 1, 1), jnp.float32),
        scratch_shapes=[
            pltpu.VMEM((_R, 1), jnp.float32),
            pltpu.VMEM((_R, 1), jnp.float32),
            pltpu.VMEM((_R, 1), jnp.float32),
        ],
        compiler_params=pltpu.CompilerParams(
            dimension_semantics=("parallel", "arbitrary"),
        ),
    )(xf, t)
    return jnp.sum(out)


# speedup vs baseline: 1.1202x; 1.0026x over previous
import functools
import math

import jax
import jax.numpy as jnp
from jax.experimental import pallas as pl
from jax.experimental.pallas import tpu as pltpu

_V = 32000
_PAD = 0
_SMOOTHING = 0.1
_CONF = 1.0 - _SMOOTHING
_EPS = _SMOOTHING / (_V - 1)
_CS = _CONF - _EPS
_C0 = (_V - 1) * _EPS * math.log(_EPS) + _CONF * math.log(_CONF)

_R = 256
_C = 16000


def _body(nc, inv_denom, x_ref, t_ref, out_ref, m_ref, s_ref, sx_ref):
    j = pl.program_id(1)
    xb = x_ref[...]
    t = t_ref[...]

    bmax = jnp.max(xb, axis=1, keepdims=True)

    t_loc = t - j * _C
    hit = jax.lax.broadcasted_iota(jnp.int32, (_R, _C), 1) == t_loc
    z_part = jnp.sum(xb * jnp.where(hit, _CONF, _EPS), axis=1, keepdims=True)

    first = j == 0
    neg_inf = jnp.full((_R, 1), -jnp.inf, dtype=jnp.float32)
    zeros = jnp.zeros((_R, 1), dtype=jnp.float32)
    m_old = jnp.where(first, neg_inf, m_ref[...])
    s_old = jnp.where(first, zeros, s_ref[...])
    z_old = jnp.where(first, zeros, sx_ref[...])

    m_new = jnp.maximum(m_old, bmax)
    s_new = s_old * jnp.exp(m_old - m_new) + jnp.sum(
        jnp.exp(xb - m_new), axis=1, keepdims=True)
    m_ref[...] = m_new
    s_ref[...] = s_new
    sx_ref[...] = z_old + z_part

    @pl.when(j == nc - 1)
    def _():
        lse = m_new + jnp.log(s_new)
        row_loss = _C0 + lse - sx_ref[...]
        valid = t != _PAD
        contrib = jnp.sum(jnp.where(valid, row_loss, 0.0)) * inv_denom
        out_ref[...] = jnp.zeros((1, 1, 1), jnp.float32) + contrib


def kernel(x, target):
    batch = x.shape[0]
    n = x.shape[0] * x.shape[1]
    xf = x.reshape(n, _V)
    t = target.reshape(n, 1).astype(jnp.int32)
    nr = n // _R
    nc = _V // _C
    out = pl.pallas_call(
        functools.partial(_body, nc, 1.0 / batch),
        grid=(nr, nc),
        in_specs=[
            pl.BlockSpec((_R, _C), lambda i, j: (i, j)),
            pl.BlockSpec((_R, 1), lambda i, j: (i, 0)),
        ],
        out_specs=pl.BlockSpec((1, 1, 1), lambda i, j: (i, 0, 0)),
        out_shape=jax.ShapeDtypeStruct((nr, 1, 1), jnp.float32),
        scratch_shapes=[
            pltpu.VMEM((_R, 1), jnp.float32),
            pltpu.VMEM((_R, 1), jnp.float32),
            pltpu.VMEM((_R, 1), jnp.float32),
        ],
        compiler_params=pltpu.CompilerParams(
            dimension_semantics=("parallel", "arbitrary"),
        ),
    )(xf, t)
    return jnp.sum(out)


# speedup vs baseline: 1.1280x; 1.0070x over previous
import functools
import math

import jax
import jax.numpy as jnp
from jax.experimental import pallas as pl
from jax.experimental.pallas import tpu as pltpu

_V = 32000
_PAD = 0
_SMOOTHING = 0.1
_CONF = 1.0 - _SMOOTHING
_EPS = _SMOOTHING / (_V - 1)
_CS = _CONF - _EPS
_C0 = (_V - 1) * _EPS * math.log(_EPS) + _CONF * math.log(_CONF)

_R = 256
_C = 16000


def _body(nc, inv_denom, x_ref, t_ref, out_ref, m_ref, s_ref, sx_ref):
    i = pl.program_id(0)
    j = pl.program_id(1)
    xb = x_ref[...]
    t = t_ref[...]

    bmax = jnp.max(xb, axis=1, keepdims=True)

    t_loc = t - j * _C
    hit = jax.lax.broadcasted_iota(jnp.int32, (_R, _C), 1) == t_loc
    z_part = jnp.sum(xb * jnp.where(hit, _CONF, _EPS), axis=1, keepdims=True)

    first = j == 0
    neg_inf = jnp.full((_R, 1), -jnp.inf, dtype=jnp.float32)
    zeros = jnp.zeros((_R, 1), dtype=jnp.float32)
    m_old = jnp.where(first, neg_inf, m_ref[...])
    s_old = jnp.where(first, zeros, s_ref[...])
    z_old = jnp.where(first, zeros, sx_ref[...])

    m_new = jnp.maximum(m_old, bmax)
    s_new = s_old * jnp.exp(m_old - m_new) + jnp.sum(
        jnp.exp(xb - m_new), axis=1, keepdims=True)
    m_ref[...] = m_new
    s_ref[...] = s_new
    sx_ref[...] = z_old + z_part

    @pl.when(j == nc - 1)
    def _():
        lse = m_new + jnp.log(s_new)
        row_loss = _C0 + lse - sx_ref[...]
        valid = t != _PAD
        contrib = jnp.sum(jnp.where(valid, row_loss, 0.0)) * inv_denom
        prev = jnp.where(i == 0, jnp.zeros((1, 1), jnp.float32), out_ref[...])
        out_ref[...] = prev + contrib


def kernel(x, target):
    batch = x.shape[0]
    n = x.shape[0] * x.shape[1]
    xf = x.reshape(n, _V)
    t = target.reshape(n, 1).astype(jnp.int32)
    nr = n // _R
    nc = _V // _C
    out = pl.pallas_call(
        functools.partial(_body, nc, 1.0 / batch),
        grid=(nr, nc),
        in_specs=[
            pl.BlockSpec((_R, _C), lambda i, j: (i, j)),
            pl.BlockSpec((_R, 1), lambda i, j: (i, 0)),
        ],
        out_specs=pl.BlockSpec((1, 1), lambda i, j: (0, 0)),
        out_shape=jax.ShapeDtypeStruct((1, 1), jnp.float32),
        scratch_shapes=[
            pltpu.VMEM((_R, 1), jnp.float32),
            pltpu.VMEM((_R, 1), jnp.float32),
            pltpu.VMEM((_R, 1), jnp.float32),
        ],
        compiler_params=pltpu.CompilerParams(
            dimension_semantics=("arbitrary", "arbitrary"),
        ),
    )(xf, t)
    return out[0, 0]


# speedup vs baseline: 1.1295x; 1.0013x over previous
import functools
import math

import jax
import jax.numpy as jnp
from jax.experimental import pallas as pl
from jax.experimental.pallas import tpu as pltpu

_V = 32000
_PAD = 0
_SMOOTHING = 0.1
_CONF = 1.0 - _SMOOTHING
_EPS = _SMOOTHING / (_V - 1)
_CS = _CONF - _EPS
_C0 = (_V - 1) * _EPS * math.log(_EPS) + _CONF * math.log(_CONF)

_R = 256
_C = 16000


def _body(nc, inv_denom, x_ref, t_ref, out_ref, m_ref, s_ref, z_ref):
    i = pl.program_id(0)
    j = pl.program_id(1)
    xb = x_ref[...]
    t = t_ref[...]

    bmax = jnp.max(xb, axis=1, keepdims=True)

    t_loc = t - j * _C
    hit = jax.lax.broadcasted_iota(jnp.int32, (_R, _C), 1) == t_loc
    z_part = jnp.sum(xb * jnp.where(hit, _CONF, _EPS), axis=1, keepdims=True)

    first = j == 0
    neg_inf = jnp.full((_R, 1), -jnp.inf, dtype=jnp.float32)
    zeros = jnp.zeros((_R, 1), dtype=jnp.float32)
    m_old = jnp.where(first, neg_inf, m_ref[...])
    s_old = jnp.where(first, zeros, s_ref[...])
    z_old = jnp.where(first, zeros, z_ref[...])

    m_new = jnp.maximum(m_old, bmax)
    s_new = s_old * jnp.exp(m_old - m_new) + jnp.sum(
        jnp.exp(xb - m_new), axis=1, keepdims=True)
    m_ref[...] = m_new
    s_ref[...] = s_new
    z_ref[...] = z_old + z_part

    @pl.when(j == nc - 1)
    def _():
        lse = m_new + jnp.log(s_new)
        row_loss = _C0 + lse - z_ref[...]
        valid = t != _PAD
        contrib = jnp.sum(jnp.where(valid, row_loss, 0.0)) * inv_denom
        prev = jnp.where(i == 0, jnp.zeros((1, 1), jnp.float32), out_ref[...])
        out_ref[...] = prev + contrib


def kernel(x, target):
    batch = x.shape[0]
    n = x.shape[0] * x.shape[1]
    xf = x.reshape(n, _V)
    t = target.reshape(n, 1).astype(jnp.int32)
    nr = n // _R
    nc = _V // _C
    out = pl.pallas_call(
        functools.partial(_body, nc, 1.0 / batch),
        grid=(nr, nc),
        in_specs=[
            pl.BlockSpec((_R, _C), lambda i, j: (i, j)),
            pl.BlockSpec((_R, 1), lambda i, j: (i, 0)),
        ],
        out_specs=pl.BlockSpec((1, 1), lambda i, j: (0, 0)),
        out_shape=jax.ShapeDtypeStruct((1, 1), jnp.float32),
        scratch_shapes=[
            pltpu.VMEM((_R, 1), jnp.float32),
            pltpu.VMEM((_R, 1), jnp.float32),
            pltpu.VMEM((_R, 1), jnp.float32),
        ],
        compiler_params=pltpu.CompilerParams(
            dimension_semantics=("arbitrary", "arbitrary"),
        ),
    )(xf, t)
    return out[0, 0]
